# baseline (speedup 1.0000x reference)
import functools

import jax
import jax.numpy as jnp
from jax import lax
from jax.experimental import pallas as pl
from jax.experimental.pallas import tpu as pltpu
from jax.experimental.pallas import tpu_sc as plsc

L = 16


def _build(B, R, N, CAP, b_per_w):
    N_PAD = ((N + L - 1) // L) * L
    CAP_PAD = ((CAP + L - 1) // L) * L
    mesh = plsc.VectorSubcoreMesh(core_axis_name="c", subcore_axis_name="s")

    @functools.partial(
        pl.kernel,
        mesh=mesh,
        out_type=jax.ShapeDtypeStruct((B,), jnp.float32),
        compiler_params=pltpu.CompilerParams(needs_layout_passes=False),
        scratch_types=[
            pltpu.VMEM((2 * b_per_w,), jnp.int32),
            pltpu.VMEM((R * N,), jnp.float32),
            pltpu.VMEM((R * N,), jnp.float32),
            pltpu.VMEM((CAP_PAD,), jnp.int32),
            pltpu.VMEM((N_PAD,), jnp.int32),
            pltpu.VMEM((b_per_w,), jnp.float32),
            pltpu.SemaphoreType.DMA,
        ],
    )
    def demand_kernel(batch_hbm, as_hbm, bs_hbm, cap_hbm, out_hbm,
                      batch_v, as_v, bs_v, cap_v, mask_v, out_v, sem):
        wid = lax.axis_index("s") * 2 + lax.axis_index("c")
        base = wid * b_per_w

        c0 = pltpu.async_copy(batch_hbm.at[pl.ds(2 * base, 2 * b_per_w)],
                              batch_v, sem)
        c1 = pltpu.async_copy(as_hbm, as_v, sem)
        c2 = pltpu.async_copy(bs_hbm, bs_v, sem)
        c3 = pltpu.async_copy(cap_hbm, cap_v.at[pl.ds(0, CAP)], sem)

        zeros = jnp.zeros((L,), jnp.int32)
        enns = jnp.full((L,), N, jnp.int32)
        lane = jax.lax.iota(jnp.int32, L)

        def zero_body(k, carry):
            mask_v[pl.ds(k * L, L)] = zeros
            return carry
        lax.fori_loop(0, N_PAD // L, zero_body, 0, unroll=8)

        c0.wait()
        c1.wait()
        c2.wait()
        c3.wait()

        for k in range(CAP_PAD // L):
            idx = cap_v[pl.ds(k * L, L)]
            if (k + 1) * L <= CAP:
                plsc.store_scatter(mask_v, [idx], enns)
            else:
                tail = jnp.full((L,), CAP - k * L, jnp.int32)
                plsc.store_scatter(mask_v, [idx], enns, mask=lane < tail)

        def body(k, carry):
            fi = (lane + k * L) * 2
            iv = plsc.load_gather(batch_v, [fi])
            jv = plsc.load_gather(batch_v, [fi + 1])
            mi = plsc.load_gather(mask_v, [iv])
            mj = plsc.load_gather(mask_v, [jv])
            off = jnp.bitwise_or(mi, mj)
            gi = iv + off
            gj = jv + off
            ai = plsc.load_gather(as_v, [gi])
            aj = plsc.load_gather(as_v, [gj])
            bi = plsc.load_gather(bs_v, [gi])
            bj = plsc.load_gather(bs_v, [gj])
            out_v[pl.ds(k * L, L)] = ai * aj + bi + bj
            return carry
        lax.fori_loop(0, b_per_w // L, body, 0, unroll=8)

        pltpu.sync_copy(out_v, out_hbm.at[pl.ds(base, b_per_w)])

    return demand_kernel


def kernel(batch, As, Bs, capital_ids):
    B = batch.shape[0]
    R, N = As.shape
    CAP = capital_ids.shape[0]
    NW = 32
    b_per_w = B // NW

    fn = _build(B, R, N, CAP, b_per_w)
    return fn(batch.reshape(-1), As.reshape(-1), Bs.reshape(-1), capital_ids)


# speedup vs baseline: 1.0900x; 1.0900x over previous
import functools

import jax
import jax.numpy as jnp
from jax import lax
from jax.experimental import pallas as pl
from jax.experimental.pallas import tpu as pltpu
from jax.experimental.pallas import tpu_sc as plsc

L = 16


def _build(B, R, N, CAP, b_per_w):
    N_PAD = ((N + L - 1) // L) * L
    CAP_PAD = ((CAP + L - 1) // L) * L
    mesh = plsc.VectorSubcoreMesh(core_axis_name="c", subcore_axis_name="s")

    @functools.partial(
        pl.kernel,
        mesh=mesh,
        out_type=jax.ShapeDtypeStruct((B,), jnp.float32),
        compiler_params=pltpu.CompilerParams(needs_layout_passes=False),
        scratch_types=[
            pltpu.VMEM((b_per_w, 2), jnp.int32),
            pltpu.VMEM((R * N,), jnp.float32),
            pltpu.VMEM((R * N,), jnp.float32),
            pltpu.VMEM((CAP_PAD,), jnp.int32),
            pltpu.VMEM((N_PAD,), jnp.int32),
            pltpu.VMEM((b_per_w,), jnp.float32),
            pltpu.SemaphoreType.DMA,
        ],
    )
    def demand_kernel(batch_hbm, as_hbm, bs_hbm, cap_hbm, out_hbm,
                      batch_v, as_v, bs_v, cap_v, mask_v, out_v, sem):
        wid = lax.axis_index("s") * 2 + lax.axis_index("c")
        base = wid * b_per_w

        copies = [
            pltpu.async_copy(batch_hbm.at[pl.ds(base, b_per_w)], batch_v, sem),
            pltpu.async_copy(as_hbm, as_v, sem),
            pltpu.async_copy(bs_hbm, bs_v, sem),
            pltpu.async_copy(cap_hbm, cap_v.at[pl.ds(0, CAP)], sem),
        ]

        zeros = jnp.zeros((L,), jnp.int32)
        ones = jnp.ones((L,), jnp.int32)
        enns = jnp.full((L,), N, jnp.int32)
        lane = jax.lax.iota(jnp.int32, L)

        def zero_body(k, carry):
            mask_v[pl.ds(k * L, L)] = zeros
            return carry
        lax.fori_loop(0, N_PAD // L, zero_body, 0, unroll=8)

        for c in copies:
            c.wait()

        for k in range(CAP_PAD // L):
            idx = cap_v[pl.ds(k * L, L)]
            if (k + 1) * L <= CAP:
                plsc.store_scatter(mask_v, [idx], enns)
            else:
                tail = jnp.full((L,), CAP - k * L, jnp.int32)
                plsc.store_scatter(mask_v, [idx], enns, mask=lane < tail)

        def body(k, carry):
            rows = lane + k * L
            iv = plsc.load_gather(batch_v, [rows, zeros])
            jv = plsc.load_gather(batch_v, [rows, ones])
            mi = plsc.load_gather(mask_v, [iv])
            mj = plsc.load_gather(mask_v, [jv])
            off = jnp.bitwise_or(mi, mj)
            gi = iv + off
            gj = jv + off
            ai = plsc.load_gather(as_v, [gi])
            aj = plsc.load_gather(as_v, [gj])
            bi = plsc.load_gather(bs_v, [gi])
            bj = plsc.load_gather(bs_v, [gj])
            out_v[pl.ds(k * L, L)] = ai * aj + bi + bj
            return carry
        lax.fori_loop(0, b_per_w // L, body, 0, unroll=8)

        pltpu.sync_copy(out_v, out_hbm.at[pl.ds(base, b_per_w)])

    return demand_kernel


def kernel(batch, As, Bs, capital_ids):
    B = batch.shape[0]
    R, N = As.shape
    CAP = capital_ids.shape[0]
    NW = 32
    b_per_w = B // NW

    fn = _build(B, R, N, CAP, b_per_w)
    return fn(batch, As.reshape(-1), Bs.reshape(-1), capital_ids)


# speedup vs baseline: 1.4647x; 1.3438x over previous
import functools

import jax
import jax.numpy as jnp
from jax import lax
from jax.experimental import pallas as pl
from jax.experimental.pallas import tpu as pltpu
from jax.experimental.pallas import tpu_sc as plsc

L = 16


def _build(B, R, N, CAP, b_per_w):
    N_PAD = ((N + L - 1) // L) * L
    CAP_PAD = ((CAP + L - 1) // L) * L
    mesh = plsc.VectorSubcoreMesh(core_axis_name="c", subcore_axis_name="s")

    @functools.partial(
        pl.kernel,
        mesh=mesh,
        out_type=jax.ShapeDtypeStruct((B,), jnp.float32),
        compiler_params=pltpu.CompilerParams(needs_layout_passes=False),
        scratch_types=[
            pltpu.VMEM((b_per_w,), jnp.int32),
            pltpu.VMEM((b_per_w,), jnp.int32),
            pltpu.VMEM((R, N), jnp.float32),
            pltpu.VMEM((R, N), jnp.float32),
            pltpu.VMEM((CAP_PAD,), jnp.int32),
            pltpu.VMEM((N_PAD,), jnp.int32),
            pltpu.VMEM((b_per_w,), jnp.float32),
            pltpu.SemaphoreType.DMA,
        ],
    )
    def demand_kernel(i_hbm, j_hbm, as_hbm, bs_hbm, cap_hbm, out_hbm,
                      i_v, j_v, as_v, bs_v, cap_v, mask_v, out_v, sem):
        wid = lax.axis_index("s") * 2 + lax.axis_index("c")
        base = wid * b_per_w

        copies = [
            pltpu.async_copy(i_hbm.at[pl.ds(base, b_per_w)], i_v, sem),
            pltpu.async_copy(j_hbm.at[pl.ds(base, b_per_w)], j_v, sem),
            pltpu.async_copy(as_hbm, as_v, sem),
            pltpu.async_copy(bs_hbm, bs_v, sem),
            pltpu.async_copy(cap_hbm, cap_v.at[pl.ds(0, CAP)], sem),
        ]

        zeros = jnp.zeros((L,), jnp.int32)
        enns = jnp.full((L,), N, jnp.int32)
        lane = jax.lax.iota(jnp.int32, L)

        def zero_body(k, carry):
            mask_v[pl.ds(k * L, L)] = zeros
            return carry
        lax.fori_loop(0, N_PAD // L, zero_body, 0, unroll=8)

        for c in copies:
            c.wait()

        for k in range(CAP_PAD // L):
            idx = cap_v[pl.ds(k * L, L)]
            if (k + 1) * L <= CAP:
                plsc.store_scatter(mask_v, [idx], enns)
            else:
                tail = jnp.full((L,), CAP - k * L, jnp.int32)
                plsc.store_scatter(mask_v, [idx], enns, mask=lane < tail)

        def body(k, carry):
            iv = i_v[pl.ds(k * L, L)]
            jv = j_v[pl.ds(k * L, L)]
            mi = plsc.load_gather(mask_v, [iv])
            mj = plsc.load_gather(mask_v, [jv])
            off = jnp.bitwise_or(mi, mj)
            gi = iv + off
            gj = jv + off
            ai = plsc.load_gather(as_v, [zeros, gi])
            aj = plsc.load_gather(as_v, [zeros, gj])
            bi = plsc.load_gather(bs_v, [zeros, gi])
            bj = plsc.load_gather(bs_v, [zeros, gj])
            out_v[pl.ds(k * L, L)] = ai * aj + bi + bj
            return carry
        lax.fori_loop(0, b_per_w // L, body, 0, unroll=8)

        pltpu.sync_copy(out_v, out_hbm.at[pl.ds(base, b_per_w)])

    return demand_kernel


def kernel(batch, As, Bs, capital_ids):
    B = batch.shape[0]
    R, N = As.shape
    CAP = capital_ids.shape[0]
    NW = 32
    b_per_w = B // NW

    fn = _build(B, R, N, CAP, b_per_w)
    return fn(batch[:, 0], batch[:, 1], As, Bs, capital_ids)
